# baseline (speedup 1.0000x reference)
import functools

import jax
import jax.numpy as jnp
from jax import lax
from jax.experimental import pallas as pl
from jax.experimental.pallas import tpu as pltpu
from jax.experimental.pallas import tpu_sc as plsc

_CHUNK = 128


def _embed_lookup(x_user2d, x_item2d, table_user, table_item, *, batch):
    info = plsc.get_sparse_core_info()
    n_workers = info.num_cores * info.num_subcores
    b_per_w = batch // n_workers
    n_chunks = b_per_w // _CHUNK
    dim = table_user.shape[1]

    mesh = plsc.VectorSubcoreMesh(core_axis_name="c", subcore_axis_name="s")

    @functools.partial(
        pl.kernel,
        mesh=mesh,
        compiler_params=pltpu.CompilerParams(use_tc_tiling_on_sc=False),
        out_type=(
            jax.ShapeDtypeStruct((batch, dim), jnp.float32),
            jax.ShapeDtypeStruct((batch, dim), jnp.float32),
        ),
        scratch_types=[
            pltpu.VMEM((n_chunks, _CHUNK), jnp.int32),
            pltpu.VMEM((n_chunks, _CHUNK), jnp.int32),
            pltpu.VMEM((b_per_w, dim), jnp.float32),
            pltpu.VMEM((b_per_w, dim), jnp.float32),
            pltpu.SemaphoreType.DMA,
            pltpu.SemaphoreType.DMA,
        ],
    )
    def k(xu_hbm, xi_hbm, tu_hbm, ti_hbm, yu_hbm, yi_hbm,
          idxu_v, idxi_v, rowsu_v, rowsi_v, semu, semi):
        wid = lax.axis_index("s") * info.num_cores + lax.axis_index("c")
        base = wid * b_per_w

        pltpu.sync_copy(xu_hbm.at[pl.ds(wid * n_chunks, n_chunks)], idxu_v)
        pltpu.sync_copy(xi_hbm.at[pl.ds(wid * n_chunks, n_chunks)], idxi_v)

        copies_u = []
        copies_i = []
        for j in range(n_chunks):
            copies_u.append(pltpu.async_copy(
                tu_hbm.at[idxu_v.at[j]],
                rowsu_v.at[pl.ds(j * _CHUNK, _CHUNK)],
                semu,
            ))
        for j in range(n_chunks):
            copies_i.append(pltpu.async_copy(
                ti_hbm.at[idxi_v.at[j]],
                rowsi_v.at[pl.ds(j * _CHUNK, _CHUNK)],
                semi,
            ))
        for c in copies_u:
            c.wait()
        pltpu.sync_copy(rowsu_v, yu_hbm.at[pl.ds(base, b_per_w)])
        for c in copies_i:
            c.wait()
        pltpu.sync_copy(rowsi_v, yi_hbm.at[pl.ds(base, b_per_w)])

    return k(x_user2d, x_item2d, table_user, table_item)


def kernel(x_user, x_item, table_user, table_item):
    batch = x_user.shape[0]
    xu = x_user.astype(jnp.int32).reshape(batch // _CHUNK, _CHUNK)
    xi = x_item.astype(jnp.int32).reshape(batch // _CHUNK, _CHUNK)
    return _embed_lookup(xu, xi, table_user, table_item, batch=batch)
